# baseline (speedup 1.0000x reference)
import jax
import jax.numpy as jnp
from jax import lax
from jax.experimental import pallas as pl
from jax.experimental.pallas import tpu as pltpu
from jax.experimental.pallas import tpu_sc as plsc

_N = 10000
_D = 128
_E = 320000
_NCLS = 64

_NC = 2
_NS = 16
_NW = _NC * _NS
_CHUNK = 64
_NBUF = 4
_C0 = 320
_C1 = 0
_STAGE = 64
_EPAD = _NS * (_C0 + _C1) * _CHUNK
_E0 = _NS * _C0 * _CHUNK
_NACC = 10112
_RPT = _NACC // _NS

_DCHUNK = 128
_DC = _EPAD // (_NW * _DCHUNK)

_MESH = plsc.VectorSubcoreMesh(core_axis_name="c", subcore_axis_name="s")


def _stream_edges(h_hbm, src_hbm, dst_hbm, sid, src_v, dst_v, rows, acc,
                  sems, nchunks):
    for off in range(0, nchunks, _STAGE):
        stage = min(_STAGE, nchunks - off)
        pltpu.sync_copy(src_hbm.at[sid].at[pl.ds(off, stage)],
                        src_v.at[pl.ds(0, stage)])
        pltpu.sync_copy(dst_hbm.at[sid].at[pl.ds(off, stage)],
                        dst_v.at[pl.ds(0, stage)])
        for b in range(_NBUF):
            pltpu.async_copy(h_hbm.at[src_v.at[b]], rows[b], sems[b])

        @pl.loop(0, stage, step=_NBUF)
        def _(j):
            for b in range(_NBUF):
                pltpu.make_async_copy(h_hbm.at[src_v.at[j + b]], rows[b],
                                      sems[b]).wait()
                pltpu.sync_copy(rows[b], acc.at[dst_v.at[j + b]], add=True)

                @pl.when(j + _NBUF + b < stage)
                def _():
                    pltpu.async_copy(h_hbm.at[src_v.at[j + _NBUF + b]],
                                     rows[b], sems[b])


def _sc_pass(h, *edge_args):
    def body(h_hbm, *hbm_refs):
        if _C1:
            src0_hbm, dst0_hbm, src1_hbm, dst1_hbm, agg_hbm, src_v, dst_v, \
                *refs = hbm_refs
        else:
            src0_hbm, dst0_hbm, agg_hbm, src_v, dst_v, *refs = hbm_refs
        rows = refs[:_NBUF]
        acc = refs[_NBUF]
        sems = refs[_NBUF + 1:]
        cid = lax.axis_index("c")
        sid = lax.axis_index("s")
        base = sid * _RPT

        @pl.loop(0, _CHUNK)
        def _(r):
            @pl.loop(0, _D // 16)
            def _(c):
                rows[0][r, pl.ds(c * 16, 16)] = jnp.zeros((16,), jnp.float32)

        nfull = _RPT // _CHUNK
        rem = _RPT - nfull * _CHUNK

        @pl.loop(0, nfull)
        def _(k):
            pltpu.sync_copy(rows[0], acc.at[pl.ds(base + k * _CHUNK, _CHUNK)])

        if rem:
            pltpu.sync_copy(rows[0].at[pl.ds(0, rem)],
                            acc.at[pl.ds(base + nfull * _CHUNK, rem)])
        plsc.subcore_barrier()

        @pl.when(cid == 0)
        def _():
            _stream_edges(h_hbm, src0_hbm, dst0_hbm, sid, src_v, dst_v,
                          rows, acc, sems, _C0)

        if _C1:
            @pl.when(cid == 1)
            def _():
                _stream_edges(h_hbm, src1_hbm, dst1_hbm, sid, src_v, dst_v,
                              rows, acc, sems, _C1)

        plsc.subcore_barrier()
        pltpu.sync_copy(acc.at[pl.ds(base, _RPT)],
                        agg_hbm.at[cid].at[pl.ds(base, _RPT)])

    f = pl.kernel(
        body,
        out_type=jax.ShapeDtypeStruct((_NC, _NACC, _D), jnp.float32),
        mesh=_MESH,
        scratch_types=(
            [pltpu.VMEM((_STAGE, _CHUNK), jnp.int32),
             pltpu.VMEM((_STAGE, _CHUNK), jnp.int32)]
            + [pltpu.VMEM((_CHUNK, _D), jnp.float32)
               for _ in range(_NBUF)]
            + [pltpu.VMEM_SHARED((_NACC, _D), jnp.float32)]
            + [pltpu.SemaphoreType.DMA for _ in range(_NBUF)]
        ),
    )
    return f(h, *edge_args)


def _sc_deg(dst_w):
    def body(dst_hbm, deg_hbm, dst_v, ones_v, dacc, sem):
        del sem
        cid = lax.axis_index("c")
        sid = lax.axis_index("s")
        wid = sid * _NC + cid
        base = sid * _RPT

        pltpu.sync_copy(dst_hbm.at[wid], dst_v)

        @pl.loop(0, _DCHUNK)
        def _(r):
            @pl.loop(0, _D // 16)
            def _(c):
                ones_v[r, pl.ds(c * 16, 16)] = jnp.zeros((16,), jnp.float32)

        @pl.loop(0, _RPT // 8)
        def _(k):
            pltpu.sync_copy(ones_v.at[pl.ds(0, 8)],
                            dacc.at[pl.ds(base + k * 8, 8)])

        @pl.loop(0, _DCHUNK)
        def _(r):
            @pl.loop(0, _D // 16)
            def _(c):
                ones_v[r, pl.ds(c * 16, 16)] = jnp.ones((16,), jnp.float32)

        plsc.subcore_barrier()

        @pl.loop(0, _DC)
        def _(j):
            pltpu.sync_copy(ones_v, dacc.at[dst_v.at[j]], add=True)

        plsc.subcore_barrier()
        pltpu.sync_copy(dacc.at[pl.ds(base, _RPT)],
                        deg_hbm.at[cid].at[pl.ds(base, _RPT)])

    f = pl.kernel(
        body,
        out_type=jax.ShapeDtypeStruct((_NC, _NACC, _D), jnp.float32),
        mesh=_MESH,
        scratch_types=[
            pltpu.VMEM((_DC, _DCHUNK), jnp.int32),
            pltpu.VMEM((_DCHUNK, _D), jnp.float32),
            pltpu.VMEM_SHARED((_NACC, _D), jnp.float32),
            pltpu.SemaphoreType.DMA,
        ],
    )
    return f(dst_w)


_BLK = 1000


def _tc_layer(h, agg_p, deg_p, w_self, w_neigh, b, fc_w=None, fc_b=None):
    n_out = _NCLS if fc_w is not None else _D
    in_specs = [
        pl.BlockSpec((_BLK, _D), lambda i: (i, 0)),
        pl.BlockSpec((_NC, _BLK, _D), lambda i: (0, i, 0)),
        pl.BlockSpec((_NC, _BLK, _D), lambda i: (0, i, 0)),
        pl.BlockSpec((_D, _D), lambda i: (0, 0)),
        pl.BlockSpec((_D, _D), lambda i: (0, 0)),
        pl.BlockSpec((1, _D), lambda i: (0, 0)),
    ]
    args = [h, agg_p, deg_p, w_self, w_neigh, b.reshape(1, _D)]
    if fc_w is not None:
        in_specs += [pl.BlockSpec((_D, _NCLS), lambda i: (0, 0)),
                     pl.BlockSpec((1, _NCLS), lambda i: (0, 0))]
        args += [fc_w, fc_b.reshape(1, _NCLS)]

    def body(h_ref, p_ref, d_ref, ws_ref, wn_ref, b_ref, *rest):
        if fc_w is not None:
            fw_ref, fb_ref, o_ref = rest
        else:
            (o_ref,) = rest
        agg = p_ref[0] + p_ref[1]
        deg = d_ref[0, :, 0:1] + d_ref[1, :, 0:1]
        hn = agg / jnp.maximum(deg, 1.0)
        y = jnp.dot(h_ref[...], ws_ref[...], preferred_element_type=jnp.float32)
        y = y + jnp.dot(hn, wn_ref[...], preferred_element_type=jnp.float32)
        y = y + b_ref[...]
        if fc_w is not None:
            y = jnp.dot(y, fw_ref[...], preferred_element_type=jnp.float32)
            y = y + fb_ref[...]
        o_ref[...] = y

    return pl.pallas_call(
        body,
        grid=(_N // _BLK,),
        in_specs=in_specs,
        out_specs=pl.BlockSpec((_BLK, n_out), lambda i: (i, 0)),
        out_shape=jax.ShapeDtypeStruct((_N, n_out), jnp.float32),
    )(*args)


def kernel(x, edge_index, W_self_0, W_neigh_0, b_0, W_self_1, W_neigh_1, b_1,
           W_self_2, W_neigh_2, b_2, fc1_W, fc1_b):
    src = edge_index[0]
    dst = edge_index[1]
    pad = _EPAD - _E
    src_p = jnp.concatenate([src, jnp.zeros((pad,), jnp.int32)])
    dst_p = jnp.concatenate([dst, jnp.full((pad,), _N, jnp.int32)])
    src0 = src_p[:_E0].reshape(_NS, _C0, _CHUNK)
    dst0 = dst_p[:_E0].reshape(_NS, _C0, _CHUNK)
    if _C1:
        src1 = src_p[_E0:].reshape(_NS, _C1, _CHUNK)
        dst1 = dst_p[_E0:].reshape(_NS, _C1, _CHUNK)
        edges = (src0, dst0, src1, dst1)
    else:
        edges = (src0, dst0)

    degp = _sc_deg(dst_p.reshape(_NW, _DC, _DCHUNK))
    p0 = _sc_pass(x, *edges)
    h1 = _tc_layer(x, p0, degp, W_self_0, W_neigh_0, b_0)
    p1 = _sc_pass(h1, *edges)
    h2 = _tc_layer(h1, p1, degp, W_self_1, W_neigh_1, b_1)
    p2 = _sc_pass(h2, *edges)
    out = _tc_layer(h2, p2, degp, W_self_2, W_neigh_2, b_2, fc1_W, fc1_b)
    return out


# speedup vs baseline: 4.1005x; 4.1005x over previous
import jax
import jax.numpy as jnp
from jax import lax
from jax.experimental import pallas as pl
from jax.experimental.pallas import tpu as pltpu
from jax.experimental.pallas import tpu_sc as plsc

_N = 10000
_D = 128
_E = 320000
_NCLS = 64

_NC = 2
_NS = 16
_NW = _NC * _NS
_CHUNK = 64
_NBUF = 4
_C0 = 160
_C1 = 160
_STAGE = 64
_EPAD = _NS * (_C0 + _C1) * _CHUNK
_E0 = _NS * _C0 * _CHUNK
_NACC = 10112
_RPT = _NACC // _NS

_DCHUNK = 128
_DC = _EPAD // (_NW * _DCHUNK)

_MESH = plsc.VectorSubcoreMesh(core_axis_name="c", subcore_axis_name="s")


def _stream_edges(h_hbm, src_hbm, dst_hbm, sid, src_v, dst_v, rows, acc,
                  sems, nchunks):
    for off in range(0, nchunks, _STAGE):
        stage = min(_STAGE, nchunks - off)
        pltpu.sync_copy(src_hbm.at[sid].at[pl.ds(off, stage)],
                        src_v.at[pl.ds(0, stage)])
        pltpu.sync_copy(dst_hbm.at[sid].at[pl.ds(off, stage)],
                        dst_v.at[pl.ds(0, stage)])
        for b in range(_NBUF):
            pltpu.async_copy(h_hbm.at[src_v.at[b]], rows[b], sems[b])

        @pl.loop(0, stage, step=_NBUF)
        def _(j):
            for b in range(_NBUF):
                pltpu.make_async_copy(h_hbm.at[src_v.at[j + b]], rows[b],
                                      sems[b]).wait()
                pltpu.sync_copy(rows[b], acc.at[dst_v.at[j + b]], add=True)

                @pl.when(j + _NBUF + b < stage)
                def _():
                    pltpu.async_copy(h_hbm.at[src_v.at[j + _NBUF + b]],
                                     rows[b], sems[b])


def _sc_pass(h, *edge_args):
    def body(h_hbm, *hbm_refs):
        if _C1:
            src0_hbm, dst0_hbm, src1_hbm, dst1_hbm, agg_hbm, src_v, dst_v, \
                *refs = hbm_refs
        else:
            src0_hbm, dst0_hbm, agg_hbm, src_v, dst_v, *refs = hbm_refs
        rows = refs[:_NBUF]
        acc = refs[_NBUF]
        sems = refs[_NBUF + 1:]
        cid = lax.axis_index("c")
        sid = lax.axis_index("s")
        base = sid * _RPT

        @pl.loop(0, _CHUNK)
        def _(r):
            @pl.loop(0, _D // 16)
            def _(c):
                rows[0][r, pl.ds(c * 16, 16)] = jnp.zeros((16,), jnp.float32)

        nfull = _RPT // _CHUNK
        rem = _RPT - nfull * _CHUNK

        @pl.loop(0, nfull)
        def _(k):
            pltpu.sync_copy(rows[0], acc.at[pl.ds(base + k * _CHUNK, _CHUNK)])

        if rem:
            pltpu.sync_copy(rows[0].at[pl.ds(0, rem)],
                            acc.at[pl.ds(base + nfull * _CHUNK, rem)])
        plsc.subcore_barrier()

        @pl.when(cid == 0)
        def _():
            _stream_edges(h_hbm, src0_hbm, dst0_hbm, sid, src_v, dst_v,
                          rows, acc, sems, _C0)

        if _C1:
            @pl.when(cid == 1)
            def _():
                _stream_edges(h_hbm, src1_hbm, dst1_hbm, sid, src_v, dst_v,
                              rows, acc, sems, _C1)

        plsc.subcore_barrier()
        pltpu.sync_copy(acc.at[pl.ds(base, _RPT)],
                        agg_hbm.at[cid].at[pl.ds(base, _RPT)])

    f = pl.kernel(
        body,
        out_type=jax.ShapeDtypeStruct((_NC, _NACC, _D), jnp.float32),
        mesh=_MESH,
        scratch_types=(
            [pltpu.VMEM((_STAGE, _CHUNK), jnp.int32),
             pltpu.VMEM((_STAGE, _CHUNK), jnp.int32)]
            + [pltpu.VMEM((_CHUNK, _D), jnp.float32)
               for _ in range(_NBUF)]
            + [pltpu.VMEM_SHARED((_NACC, _D), jnp.float32)]
            + [pltpu.SemaphoreType.DMA for _ in range(_NBUF)]
        ),
    )
    return f(h, *edge_args)


def _sc_deg(dst_w):
    def body(dst_hbm, deg_hbm, dst_v, ones_v, dacc, sem):
        del sem
        cid = lax.axis_index("c")
        sid = lax.axis_index("s")
        wid = sid * _NC + cid
        base = sid * _RPT

        pltpu.sync_copy(dst_hbm.at[wid], dst_v)

        @pl.loop(0, _DCHUNK)
        def _(r):
            @pl.loop(0, _D // 16)
            def _(c):
                ones_v[r, pl.ds(c * 16, 16)] = jnp.zeros((16,), jnp.float32)

        @pl.loop(0, _RPT // 8)
        def _(k):
            pltpu.sync_copy(ones_v.at[pl.ds(0, 8)],
                            dacc.at[pl.ds(base + k * 8, 8)])

        @pl.loop(0, _DCHUNK)
        def _(r):
            @pl.loop(0, _D // 16)
            def _(c):
                ones_v[r, pl.ds(c * 16, 16)] = jnp.ones((16,), jnp.float32)

        plsc.subcore_barrier()

        @pl.loop(0, _DC)
        def _(j):
            pltpu.sync_copy(ones_v, dacc.at[dst_v.at[j]], add=True)

        plsc.subcore_barrier()
        pltpu.sync_copy(dacc.at[pl.ds(base, _RPT)],
                        deg_hbm.at[cid].at[pl.ds(base, _RPT)])

    f = pl.kernel(
        body,
        out_type=jax.ShapeDtypeStruct((_NC, _NACC, _D), jnp.float32),
        mesh=_MESH,
        scratch_types=[
            pltpu.VMEM((_DC, _DCHUNK), jnp.int32),
            pltpu.VMEM((_DCHUNK, _D), jnp.float32),
            pltpu.VMEM_SHARED((_NACC, _D), jnp.float32),
            pltpu.SemaphoreType.DMA,
        ],
    )
    return f(dst_w)


_BLK = 1000


def _tc_layer(h, agg_p, deg_p, w_self, w_neigh, b, fc_w=None, fc_b=None):
    n_out = _NCLS if fc_w is not None else _D
    in_specs = [
        pl.BlockSpec((_BLK, _D), lambda i: (i, 0)),
        pl.BlockSpec((_NC, _BLK, _D), lambda i: (0, i, 0)),
        pl.BlockSpec((_NC, _BLK, _D), lambda i: (0, i, 0)),
        pl.BlockSpec((_D, _D), lambda i: (0, 0)),
        pl.BlockSpec((_D, _D), lambda i: (0, 0)),
        pl.BlockSpec((1, _D), lambda i: (0, 0)),
    ]
    args = [h, agg_p, deg_p, w_self, w_neigh, b.reshape(1, _D)]
    if fc_w is not None:
        in_specs += [pl.BlockSpec((_D, _NCLS), lambda i: (0, 0)),
                     pl.BlockSpec((1, _NCLS), lambda i: (0, 0))]
        args += [fc_w, fc_b.reshape(1, _NCLS)]

    def body(h_ref, p_ref, d_ref, ws_ref, wn_ref, b_ref, *rest):
        if fc_w is not None:
            fw_ref, fb_ref, o_ref = rest
        else:
            (o_ref,) = rest
        agg = p_ref[0] + p_ref[1]
        deg = d_ref[0, :, 0:1] + d_ref[1, :, 0:1]
        hn = agg / jnp.maximum(deg, 1.0)
        y = jnp.dot(h_ref[...], ws_ref[...], preferred_element_type=jnp.float32)
        y = y + jnp.dot(hn, wn_ref[...], preferred_element_type=jnp.float32)
        y = y + b_ref[...]
        if fc_w is not None:
            y = jnp.dot(y, fw_ref[...], preferred_element_type=jnp.float32)
            y = y + fb_ref[...]
        o_ref[...] = y

    return pl.pallas_call(
        body,
        grid=(_N // _BLK,),
        in_specs=in_specs,
        out_specs=pl.BlockSpec((_BLK, n_out), lambda i: (i, 0)),
        out_shape=jax.ShapeDtypeStruct((_N, n_out), jnp.float32),
    )(*args)


def kernel(x, edge_index, W_self_0, W_neigh_0, b_0, W_self_1, W_neigh_1, b_1,
           W_self_2, W_neigh_2, b_2, fc1_W, fc1_b):
    src = edge_index[0]
    dst = edge_index[1]
    pad = _EPAD - _E
    pad_iota = lax.iota(jnp.int32, pad)
    src_p = jnp.concatenate([src, pad_iota % _N])
    dst_p = jnp.concatenate([dst, _N + pad_iota % (_NACC - _N)])
    src0 = src_p[:_E0].reshape(_NS, _C0, _CHUNK)
    dst0 = dst_p[:_E0].reshape(_NS, _C0, _CHUNK)
    if _C1:
        src1 = src_p[_E0:].reshape(_NS, _C1, _CHUNK)
        dst1 = dst_p[_E0:].reshape(_NS, _C1, _CHUNK)
        edges = (src0, dst0, src1, dst1)
    else:
        edges = (src0, dst0)

    degp = _sc_deg(dst_p.reshape(_NW, _DC, _DCHUNK))
    p0 = _sc_pass(x, *edges)
    h1 = _tc_layer(x, p0, degp, W_self_0, W_neigh_0, b_0)
    p1 = _sc_pass(h1, *edges)
    h2 = _tc_layer(h1, p1, degp, W_self_1, W_neigh_1, b_1)
    p2 = _sc_pass(h2, *edges)
    out = _tc_layer(h2, p2, degp, W_self_2, W_neigh_2, b_2, fc1_W, fc1_b)
    return out


# speedup vs baseline: 4.9156x; 1.1988x over previous
import dataclasses

import jax
import jax.numpy as jnp
from jax import lax
from jax.experimental import pallas as pl
from jax.experimental.pallas import tpu as pltpu
from jax.experimental.pallas import tpu_sc as plsc

_N = 10000
_D = 128
_E = 320000
_NCLS = 64

_NC = 2
_NS = 16
_NW = _NC * _NS
_CHUNK = 64
_NBUF = 4
_CW = 160
_STAGE = 64
_EPAD = _NW * _CW * _CHUNK
_NACC = 10112
_RPT = _NACC // _NS

_MESH = plsc.VectorSubcoreMesh(core_axis_name="c", subcore_axis_name="s")


def _stream_edges(h_hbm, src_hbm, dst_hbm, wid, src_v, dst_v, rows, acc,
                  sems):
    for off in range(0, _CW, _STAGE):
        stage = min(_STAGE, _CW - off)
        pltpu.sync_copy(src_hbm.at[wid].at[pl.ds(off, stage)],
                        src_v.at[pl.ds(0, stage)])
        pltpu.sync_copy(dst_hbm.at[wid].at[pl.ds(off, stage)],
                        dst_v.at[pl.ds(0, stage)])
        for b in range(_NBUF):
            pltpu.async_copy(h_hbm.at[src_v.at[b]], rows[b], sems[b])

        @pl.loop(0, stage, step=_NBUF)
        def _(j):
            for b in range(_NBUF):
                pltpu.make_async_copy(h_hbm.at[src_v.at[j + b]], rows[b],
                                      sems[b]).wait()
                pltpu.sync_copy(rows[b], acc.at[dst_v.at[j + b]], add=True)

                @pl.when(j + _NBUF + b < stage)
                def _():
                    pltpu.async_copy(h_hbm.at[src_v.at[j + _NBUF + b]],
                                     rows[b], sems[b])


def _sc_pass(h, src_w, dst_w):
    def body(h_hbm, src_hbm, dst_hbm, agg_hbm, src_v, dst_v, *rest):
        rows = rest[:_NBUF]
        acc = rest[_NBUF]
        sems = rest[_NBUF + 1:]
        cid = lax.axis_index("c")
        sid = lax.axis_index("s")
        wid = sid * _NC + cid
        base = sid * _RPT

        @pl.loop(0, _CHUNK)
        def _(r):
            @pl.loop(0, _D // 16)
            def _(c):
                rows[0][r, pl.ds(c * 16, 16)] = jnp.zeros((16,), jnp.float32)

        nfull = _RPT // _CHUNK
        rem = _RPT - nfull * _CHUNK

        @pl.loop(0, nfull)
        def _(k):
            pltpu.sync_copy(rows[0], acc.at[pl.ds(base + k * _CHUNK, _CHUNK)])

        if rem:
            pltpu.sync_copy(rows[0].at[pl.ds(0, rem)],
                            acc.at[pl.ds(base + nfull * _CHUNK, rem)])

        plsc.subcore_barrier()

        _stream_edges(h_hbm, src_hbm, dst_hbm, wid, src_v, dst_v,
                      rows, acc, sems)

        plsc.subcore_barrier()
        pltpu.sync_copy(acc.at[pl.ds(base, _RPT)],
                        agg_hbm.at[cid].at[pl.ds(base, _RPT)])

    f = pl.kernel(
        body,
        out_type=jax.ShapeDtypeStruct((_NC, _NACC, _D), jnp.float32),
        mesh=_MESH,
        scratch_types=(
            [pltpu.VMEM((_STAGE, _CHUNK), jnp.int32),
             pltpu.VMEM((_STAGE, _CHUNK), jnp.int32)]
            + [pltpu.VMEM((_CHUNK, _D), jnp.float32)
               for _ in range(_NBUF)]
            + [pltpu.VMEM_SHARED((_NACC, _D), jnp.float32)]
            + [pltpu.SemaphoreType.DMA for _ in range(_NBUF)]
        ),
    )
    return f(h, src_w, dst_w)


def _sc_deg(dst_flat):
    epw = _CW * _CHUNK

    def body(dst_hbm, deg_hbm, dst_v, hist):
        cid = lax.axis_index("c")
        sid = lax.axis_index("s")
        wid = sid * _NC + cid

        pltpu.sync_copy(dst_hbm.at[wid], dst_v)

        @pl.loop(0, _NACC // 16)
        def _(k):
            hist[pl.ds(k * 16, 16)] = jnp.zeros((16,), jnp.float32)

        @pl.loop(0, epw // 16)
        def _(i):
            vals = dst_v[pl.ds(i * 16, 16)]
            plsc.addupdate_scatter(hist, [vals], jnp.ones((16,), jnp.float32))

        pltpu.sync_copy(hist, deg_hbm.at[wid])

    cp = pltpu.CompilerParams()
    if "needs_layout_passes" in pltpu.CompilerParams.__dataclass_fields__:
        cp = dataclasses.replace(cp, needs_layout_passes=False)
    f = pl.kernel(
        body,
        out_type=jax.ShapeDtypeStruct((_NW, _NACC), jnp.float32),
        mesh=_MESH,
        scratch_types=[
            pltpu.VMEM((epw,), jnp.int32),
            pltpu.VMEM((_NACC,), jnp.float32),
        ],
        compiler_params=cp,
    )
    return f(dst_flat)


_BLK = 1000


def _tc_layer(h, agg_p, deg_p, w_self, w_neigh, b, fc_w=None, fc_b=None):
    n_out = _NCLS if fc_w is not None else _D
    in_specs = [
        pl.BlockSpec((_BLK, _D), lambda i: (i, 0)),
        pl.BlockSpec((_NC, _BLK, _D), lambda i: (0, i, 0)),
        pl.BlockSpec((_BLK, _NW), lambda i: (i, 0)),
        pl.BlockSpec((_D, _D), lambda i: (0, 0)),
        pl.BlockSpec((_D, _D), lambda i: (0, 0)),
        pl.BlockSpec((1, _D), lambda i: (0, 0)),
    ]
    args = [h, agg_p, deg_p, w_self, w_neigh, b.reshape(1, _D)]
    if fc_w is not None:
        in_specs += [pl.BlockSpec((_D, _NCLS), lambda i: (0, 0)),
                     pl.BlockSpec((1, _NCLS), lambda i: (0, 0))]
        args += [fc_w, fc_b.reshape(1, _NCLS)]

    def body(h_ref, p_ref, d_ref, ws_ref, wn_ref, b_ref, *rest):
        if fc_w is not None:
            fw_ref, fb_ref, o_ref = rest
        else:
            (o_ref,) = rest
        agg = p_ref[0] + p_ref[1]
        deg = jnp.sum(d_ref[...], axis=1)[:, None]
        hn = agg / jnp.maximum(deg, 1.0)
        y = jnp.dot(h_ref[...], ws_ref[...], preferred_element_type=jnp.float32)
        y = y + jnp.dot(hn, wn_ref[...], preferred_element_type=jnp.float32)
        y = y + b_ref[...]
        if fc_w is not None:
            y = jnp.dot(y, fw_ref[...], preferred_element_type=jnp.float32)
            y = y + fb_ref[...]
        o_ref[...] = y

    return pl.pallas_call(
        body,
        grid=(_N // _BLK,),
        in_specs=in_specs,
        out_specs=pl.BlockSpec((_BLK, n_out), lambda i: (i, 0)),
        out_shape=jax.ShapeDtypeStruct((_N, n_out), jnp.float32),
    )(*args)


def kernel(x, edge_index, W_self_0, W_neigh_0, b_0, W_self_1, W_neigh_1, b_1,
           W_self_2, W_neigh_2, b_2, fc1_W, fc1_b):
    src = edge_index[0]
    dst = edge_index[1]
    pad = _EPAD - _E
    pad_iota = lax.iota(jnp.int32, pad)
    src_p = jnp.concatenate([src, pad_iota % _N])
    dst_p = jnp.concatenate([dst, _N + pad_iota % (_NACC - _N)])
    src_w = src_p.reshape(_NW, _CW, _CHUNK)
    dst_w = dst_p.reshape(_NW, _CW, _CHUNK)

    degp = _sc_deg(dst_p.reshape(_NW, _CW * _CHUNK))
    degp = jnp.transpose(degp)
    p0 = _sc_pass(x, src_w, dst_w)
    h1 = _tc_layer(x, p0, degp, W_self_0, W_neigh_0, b_0)
    p1 = _sc_pass(h1, src_w, dst_w)
    h2 = _tc_layer(h1, p1, degp, W_self_1, W_neigh_1, b_1)
    p2 = _sc_pass(h2, src_w, dst_w)
    out = _tc_layer(h2, p2, degp, W_self_2, W_neigh_2, b_2, fc1_W, fc1_b)
    return out


# speedup vs baseline: 5.0343x; 1.0241x over previous
import dataclasses

import jax
import jax.numpy as jnp
from jax import lax
from jax.experimental import pallas as pl
from jax.experimental.pallas import tpu as pltpu
from jax.experimental.pallas import tpu_sc as plsc

_N = 10000
_D = 128
_E = 320000
_NCLS = 64

_NC = 2
_NS = 16
_NW = _NC * _NS
_CHUNK = 64
_NBUF = 4
_CW = 160
_STAGE = 64
_EPAD = _NW * _CW * _CHUNK
_NACC = 10112
_RPT = _NACC // _NS

_MESH = plsc.VectorSubcoreMesh(core_axis_name="c", subcore_axis_name="s")


def _stream_edges(h_hbm, src_hbm, dst_hbm, wid, src_v, dst_v, rows, acc,
                  sems):
    for off in range(0, _CW, _STAGE):
        stage = min(_STAGE, _CW - off)
        pltpu.sync_copy(src_hbm.at[wid].at[pl.ds(off, stage)],
                        src_v.at[pl.ds(0, stage)])
        pltpu.sync_copy(dst_hbm.at[wid].at[pl.ds(off, stage)],
                        dst_v.at[pl.ds(0, stage)])
        for b in range(_NBUF):
            pltpu.async_copy(h_hbm.at[src_v.at[b]], rows[b], sems[b])

        @pl.loop(0, stage, step=_NBUF)
        def _(j):
            for b in range(_NBUF):
                pltpu.make_async_copy(h_hbm.at[src_v.at[j + b]], rows[b],
                                      sems[b]).wait()
                pltpu.sync_copy(rows[b], acc.at[dst_v.at[j + b]], add=True)

                @pl.when(j + _NBUF + b < stage)
                def _():
                    pltpu.async_copy(h_hbm.at[src_v.at[j + _NBUF + b]],
                                     rows[b], sems[b])


def _sc_pass(h, src_w, dst_w):
    def body(h_hbm, src_hbm, dst_hbm, agg_hbm, src_v, dst_v, *rest):
        rows = rest[:_NBUF]
        acc = rest[_NBUF]
        sems = rest[_NBUF + 1:]
        cid = lax.axis_index("c")
        sid = lax.axis_index("s")
        wid = sid * _NC + cid
        base = sid * _RPT

        @pl.loop(0, _CHUNK)
        def _(r):
            @pl.loop(0, _D // 16)
            def _(c):
                rows[0][r, pl.ds(c * 16, 16)] = jnp.zeros((16,), jnp.float32)

        nfull = _RPT // _CHUNK
        rem = _RPT - nfull * _CHUNK

        @pl.loop(0, nfull)
        def _(k):
            pltpu.sync_copy(rows[0], acc.at[pl.ds(base + k * _CHUNK, _CHUNK)])

        if rem:
            pltpu.sync_copy(rows[0].at[pl.ds(0, rem)],
                            acc.at[pl.ds(base + nfull * _CHUNK, rem)])

        plsc.subcore_barrier()

        _stream_edges(h_hbm, src_hbm, dst_hbm, wid, src_v, dst_v,
                      rows, acc, sems)

        plsc.subcore_barrier()
        pltpu.sync_copy(acc.at[pl.ds(base, _RPT)],
                        agg_hbm.at[cid].at[pl.ds(base, _RPT)])

    f = pl.kernel(
        body,
        out_type=jax.ShapeDtypeStruct((_NC, _NACC, _D), jnp.float32),
        mesh=_MESH,
        scratch_types=(
            [pltpu.VMEM((_STAGE, _CHUNK), jnp.int32),
             pltpu.VMEM((_STAGE, _CHUNK), jnp.int32)]
            + [pltpu.VMEM((_CHUNK, _D), jnp.float32)
               for _ in range(_NBUF)]
            + [pltpu.VMEM_SHARED((_NACC, _D), jnp.float32)]
            + [pltpu.SemaphoreType.DMA for _ in range(_NBUF)]
        ),
    )
    return f(h, src_w, dst_w)


def _sc_deg(dst_flat):
    epw = _CW * _CHUNK

    def body(dst_hbm, deg_hbm, dst_v, hist):
        cid = lax.axis_index("c")
        sid = lax.axis_index("s")
        wid = sid * _NC + cid

        pltpu.sync_copy(dst_hbm.at[wid], dst_v)

        @pl.loop(0, _NACC // 16)
        def _(k):
            hist[pl.ds(k * 16, 16)] = jnp.zeros((16,), jnp.float32)

        @pl.loop(0, epw // 16)
        def _(i):
            vals = dst_v[pl.ds(i * 16, 16)]
            plsc.addupdate_scatter(hist, [vals], jnp.ones((16,), jnp.float32))

        pltpu.sync_copy(hist, deg_hbm.at[wid])

    cp = pltpu.CompilerParams()
    if "needs_layout_passes" in pltpu.CompilerParams.__dataclass_fields__:
        cp = dataclasses.replace(cp, needs_layout_passes=False)
    f = pl.kernel(
        body,
        out_type=jax.ShapeDtypeStruct((_NW, _NACC), jnp.float32),
        mesh=_MESH,
        scratch_types=[
            pltpu.VMEM((epw,), jnp.int32),
            pltpu.VMEM((_NACC,), jnp.float32),
        ],
        compiler_params=cp,
    )
    return f(dst_flat)


_BLK = 2000


def _tc_layer(h, agg_p, deg_p, w_self, w_neigh, b, fc_w=None, fc_b=None):
    n_out = _NCLS if fc_w is not None else _D
    in_specs = [
        pl.BlockSpec((_BLK, _D), lambda i: (i, 0)),
        pl.BlockSpec((_NC, _BLK, _D), lambda i: (0, i, 0)),
        pl.BlockSpec((_BLK, _NW), lambda i: (i, 0)),
        pl.BlockSpec((_D, _D), lambda i: (0, 0)),
        pl.BlockSpec((_D, _D), lambda i: (0, 0)),
        pl.BlockSpec((1, _D), lambda i: (0, 0)),
    ]
    args = [h, agg_p, deg_p, w_self, w_neigh, b.reshape(1, _D)]
    if fc_w is not None:
        in_specs += [pl.BlockSpec((_D, _NCLS), lambda i: (0, 0)),
                     pl.BlockSpec((1, _NCLS), lambda i: (0, 0))]
        args += [fc_w, fc_b.reshape(1, _NCLS)]

    def body(h_ref, p_ref, d_ref, ws_ref, wn_ref, b_ref, *rest):
        if fc_w is not None:
            fw_ref, fb_ref, o_ref = rest
        else:
            (o_ref,) = rest
        agg = p_ref[0] + p_ref[1]
        deg = jnp.sum(d_ref[...], axis=1)[:, None]
        hn = agg / jnp.maximum(deg, 1.0)
        y = jnp.dot(h_ref[...], ws_ref[...], preferred_element_type=jnp.float32)
        y = y + jnp.dot(hn, wn_ref[...], preferred_element_type=jnp.float32)
        y = y + b_ref[...]
        if fc_w is not None:
            y = jnp.dot(y, fw_ref[...], preferred_element_type=jnp.float32)
            y = y + fb_ref[...]
        o_ref[...] = y

    return pl.pallas_call(
        body,
        grid=(_N // _BLK,),
        in_specs=in_specs,
        out_specs=pl.BlockSpec((_BLK, n_out), lambda i: (i, 0)),
        out_shape=jax.ShapeDtypeStruct((_N, n_out), jnp.float32),
    )(*args)


def kernel(x, edge_index, W_self_0, W_neigh_0, b_0, W_self_1, W_neigh_1, b_1,
           W_self_2, W_neigh_2, b_2, fc1_W, fc1_b):
    src = edge_index[0]
    dst = edge_index[1]
    pad = _EPAD - _E
    pad_iota = lax.iota(jnp.int32, pad)
    src_p = jnp.concatenate([src, pad_iota % _N])
    dst_p = jnp.concatenate([dst, _N + pad_iota % (_NACC - _N)])
    src_w = src_p.reshape(_NW, _CW, _CHUNK)
    dst_w = dst_p.reshape(_NW, _CW, _CHUNK)

    degp = _sc_deg(dst_p.reshape(_NW, _CW * _CHUNK))
    degp = jnp.transpose(degp)
    p0 = _sc_pass(x, src_w, dst_w)
    h1 = _tc_layer(x, p0, degp, W_self_0, W_neigh_0, b_0)
    p1 = _sc_pass(h1, src_w, dst_w)
    h2 = _tc_layer(h1, p1, degp, W_self_1, W_neigh_1, b_1)
    p2 = _sc_pass(h2, src_w, dst_w)
    out = _tc_layer(h2, p2, degp, W_self_2, W_neigh_2, b_2, fc1_W, fc1_b)
    return out
